# baseline (speedup 1.0000x reference)
import jax
import jax.numpy as jnp
from jax.experimental import pallas as pl


def _sigmoid_body(x_ref, o_ref):
    o_ref[...] = jax.nn.sigmoid(x_ref[...])


def kernel(pred_logits, pred_boxes, pred_angles, target_sizes):
    b = pred_logits.shape[0]
    num_classes = pred_logits.shape[2]
    prob = pl.pallas_call(
        _sigmoid_body,
        out_shape=jax.ShapeDtypeStruct(pred_logits.shape, pred_logits.dtype),
    )(pred_logits)
    topk_values, topk_indexes = jax.lax.top_k(prob.reshape(b, -1), 400)
    scores = topk_values
    topk_boxes = topk_indexes // num_classes
    labels = topk_indexes % num_classes
    bidx = jnp.arange(b)[:, None]
    polys = pred_boxes[bidx, topk_boxes]
    angles = pred_angles[bidx, topk_boxes]
    theta_pred = jnp.argmax(jax.nn.sigmoid(angles), axis=2, keepdims=True)
    theta_pred = jnp.concatenate(
        [theta_pred, (theta_pred + 90) % 360, (theta_pred + 180) % 360,
         (theta_pred + 270) % 360], axis=-1)
    img_w = target_sizes[:, 1]
    scale_fct = jnp.stack([img_w] * 13, axis=1)
    polys = polys * scale_fct[:, None, :, None]
    valid_mask = scores > 0.005
    return (scores, labels, polys, theta_pred, valid_mask)


# speedup vs baseline: 1.1720x; 1.1720x over previous
import functools

import jax
import jax.numpy as jnp
from jax import lax
from jax.experimental import pallas as pl
from jax.experimental.pallas import tpu as pltpu
from jax.experimental.pallas import tpu_sc as plsc

BB = 8
NN = 5000
CC = 16
KK = 400
NSUB = 16
NPER = NN * CC // NSUB
SHARD = 5120
NV = SHARD // 16
PADP = 0x7FFFFFFF
ROWS = 32
NROW = KK // NSUB

_i32 = jnp.int32
_f32 = jnp.float32


def _sc_body(c_hbm, boxes_hbm, angles_hbm, ts_hbm,
             scores_out, labels_out, theta_out, polys_out,
             c_v, candA_c, candA_p, candB_c, candB_p, tie_p,
             above_c, above_p, hist_v, idx256, uni_c, uni_p, uni2_c, uni2_p,
             base_v, cntrow, ts_v, srt_c, srt_p, idxg, boxrows, angrows,
             sc_st, lb_st, th_st, cnt_smem, asm_a, asm_b, fin_c, fin_p,
             sp_hist, sp_cnt, sp_above_c, sp_above_p, sp_tie_p,
             sp_sorted_c, sp_sorted_p, sem_a, sem_b):
  core = lax.axis_index("c")
  sidx = lax.axis_index("s")
  iota = lax.iota(_i32, 16)

  def pc(m):
    return jnp.sum(m.astype(_i32))

  occ0, _ = plsc.scan_count(jnp.zeros((16,), _i32))
  base0 = jnp.max(occ0) - 15
  cnt_corr = 1 - base0

  pltpu.sync_copy(ts_hbm, ts_v)

  def batch_body(bi, _):
    b = core * 4 + bi

    pltpu.sync_copy(c_hbm.at[b, sidx], c_v)

    def zh(j, _):
      hist_v[pl.ds(j * 16, 16)] = jnp.zeros((16,), _i32)
      return 0
    lax.fori_loop(0, 16, zh, 0)

    @pl.when(sidx == 0)
    def _():
      for l in range(4):
        pltpu.sync_copy(hist_v, sp_hist.at[pl.ds(256 * l, 256)])
    plsc.subcore_barrier()

    rank = jnp.int32(KK)
    t_acc = jnp.int32(-2147483648)
    acnt = jnp.int32(0)
    ccnt = jnp.int32(0)
    shifts = [23, 15, 7, 0]
    masks = [0xFF, 0xFF, 0xFF, 0x7F]
    srcs = [(c_v, None), (candA_c, candA_p), (candB_c, candB_p),
            (candA_c, candA_p)]
    dsts = [(candA_c, candA_p), (candB_c, candB_p), (candA_c, candA_p),
            (candB_c, candB_p)]

    for l in range(4):
      sh, mk = shifts[l], masks[l]
      src_c, src_p = srcs[l]
      dst_c, dst_p = dsts[l]
      trips = NV if l == 0 else (ccnt + 15) >> 4

      def dig(v):
        return (lax.shift_right_logical(v, sh) & mk).astype(_i32)

      def zh2(j, _):
        hist_v[pl.ds(j * 16, 16)] = jnp.zeros((16,), _i32)
        return 0
      lax.fori_loop(0, 16, zh2, 0)

      if l == 0:
        def hb(i, _):
          v = src_c[pl.ds(i * 16, 16)]
          d = dig(v)
          occ, lastm = plsc.scan_count(d)
          plsc.addupdate_scatter(hist_v, [d], occ + cnt_corr, mask=lastm)
          return 0
      else:
        def hb(i, _, _cc=ccnt, _src=src_c, _dig=dig):
          v = _src[pl.ds(i * 16, 16)]
          valid = (i * 16 + iota) < _cc
          d = _dig(v)
          occ, lastm = plsc.scan_count(d, mask=valid)
          plsc.addupdate_scatter(hist_v, [d], occ + cnt_corr, mask=lastm)
          return 0
      lax.fori_loop(0, trips, hb, 0)

      def wi(j, _):
        idx256[pl.ds(j * 16, 16)] = 256 * l + j * 16 + iota
        return 0
      lax.fori_loop(0, 16, wi, 0)
      pltpu.sync_copy(hist_v, sp_hist.at[idx256], add=True)
      plsc.subcore_barrier()
      pltpu.sync_copy(sp_hist.at[pl.ds(256 * l, 256)], hist_v)

      def sb(j, carry):
        run, nlt, abv = carry
        g = hist_v[pl.ds(j * 16, 16)]
        cum = plsc.cumsum(g) + run
        m = cum < rank
        return (run + jnp.sum(g), nlt + pc(m),
                jnp.maximum(abv, jnp.max(jnp.where(m, cum, 0))))
      _, bl, abv = lax.fori_loop(0, 16, sb, (jnp.int32(0), jnp.int32(0),
                                             jnp.int32(0)))
      rank = rank - abv
      t_acc = t_acc | lax.shift_left(bl, sh)

      if l == 0:
        def cb(i, carry, _bl=bl, _dig=dig):
          ac, cc = carry
          v = c_v[pl.ds(i * 16, 16)]
          lp = i * 16 + iota
          p = jnp.where(lp < NPER, sidx * NPER + lp, PADP)
          d = _dig(v)
          m_ab = d < _bl
          m_in = d == _bl
          plsc.store_compressed(above_c.at[pl.ds(ac, 16)], v, mask=m_ab)
          plsc.store_compressed(above_p.at[pl.ds(ac, 16)], p, mask=m_ab)
          plsc.store_compressed(candA_c.at[pl.ds(cc, 16)], v, mask=m_in)
          plsc.store_compressed(candA_p.at[pl.ds(cc, 16)], p, mask=m_in)
          return ac + pc(m_ab), cc + pc(m_in)
      else:
        def cb(i, carry, _bl=bl, _cc=ccnt, _src_c=src_c, _src_p=src_p,
               _dst_c=dst_c, _dst_p=dst_p, _dig=dig):
          ac, cc = carry
          v = _src_c[pl.ds(i * 16, 16)]
          p = _src_p[pl.ds(i * 16, 16)]
          valid = (i * 16 + iota) < _cc
          d = _dig(v)
          m_ab = (d < _bl) & valid
          m_in = (d == _bl) & valid
          plsc.store_compressed(above_c.at[pl.ds(ac, 16)], v, mask=m_ab)
          plsc.store_compressed(above_p.at[pl.ds(ac, 16)], p, mask=m_ab)
          plsc.store_compressed(_dst_c.at[pl.ds(cc, 16)], v, mask=m_in)
          plsc.store_compressed(_dst_p.at[pl.ds(cc, 16)], p, mask=m_in)
          return ac + pc(m_ab), cc + pc(m_in)
      acnt, ccnt = lax.fori_loop(0, trips, cb, (acnt, jnp.int32(0)))

    def tb(i, tc):
      p = candB_p[pl.ds(i * 16, 16)]
      m = ((i * 16 + iota) < ccnt) & (p != PADP)
      plsc.store_compressed(tie_p.at[pl.ds(tc, 16)], p, mask=m)
      return tc + pc(m)
    tcnt = lax.fori_loop(0, (ccnt + 15) >> 4, tb, jnp.int32(0))
    tpub = jnp.minimum(tcnt, 512)

    cntrow[...] = jnp.where(iota == 0, acnt, jnp.where(iota == 1, tpub, 0))
    pltpu.sync_copy(cntrow, sp_cnt.at[sidx])
    for k in range(4):
      @pl.when(acnt > 128 * k)
      def _(k=k):
        pltpu.sync_copy(above_c.at[pl.ds(128 * k, 128)],
                        sp_above_c.at[sidx, pl.ds(128 * k, 128)])
        pltpu.sync_copy(above_p.at[pl.ds(128 * k, 128)],
                        sp_above_p.at[sidx, pl.ds(128 * k, 128)])
      @pl.when(tpub > 128 * k)
      def _(k=k):
        pltpu.sync_copy(tie_p.at[pl.ds(128 * k, 128)],
                        sp_tie_p.at[sidx, pl.ds(128 * k, 128)])
    plsc.subcore_barrier()

    @pl.when(sidx == 0)
    def _():
      pltpu.sync_copy(sp_cnt, cnt_smem)
      def zi(j, _):
        uni_c[pl.ds(j * 16, 16)] = jnp.full((16,), -1, _i32)
        uni_p[pl.ds(j * 16, 16)] = jnp.zeros((16,), _i32)
        uni2_c[pl.ds(j * 16, 16)] = jnp.full((16,), -1, _i32)
        uni2_p[pl.ds(j * 16, 16)] = jnp.zeros((16,), _i32)
        return 0
      lax.fori_loop(0, 32, zi, 0)

      off = jnp.int32(0)
      for w in range(16):
        ca = cnt_smem[w, 0]
        for k in range(4):
          @pl.when(ca > 128 * k)
          def _(k=k, w=w):
            pltpu.sync_copy(sp_above_c.at[w, pl.ds(128 * k, 128)],
                            asm_a.at[pl.ds(128 * k, 128)])
            pltpu.sync_copy(sp_above_p.at[w, pl.ds(128 * k, 128)],
                            asm_b.at[pl.ds(128 * k, 128)])
        def ab(i, o, _ca=ca):
          m = (i * 16 + iota) < _ca
          plsc.store_compressed(uni_c.at[pl.ds(o, 16)],
                                asm_a[pl.ds(i * 16, 16)], mask=m)
          plsc.store_compressed(uni_p.at[pl.ds(o, 16)],
                                asm_b[pl.ds(i * 16, 16)], mask=m)
          return o + pc(m)
        off = lax.fori_loop(0, (ca + 15) >> 4, ab, off)

      trips_s = (off + 15) >> 4
      bufs = [(uni_c, uni_p, uni2_c, uni2_p), (uni2_c, uni2_p, uni_c, uni_p)]
      for pi, sh in enumerate([0, 8, 16, 24]):
        s_c, s_p, d_c, d_p = bufs[pi % 2]

        def zb(j, _):
          base_v[pl.ds(j * 16, 16)] = jnp.zeros((16,), _i32)
          return 0
        lax.fori_loop(0, 16, zb, 0)

        def shb(i, _, _s_c=s_c, _sh=sh):
          v = _s_c[pl.ds(i * 16, 16)]
          d = (lax.shift_right_logical(v, _sh) & 0xFF).astype(_i32)
          occ, lastm = plsc.scan_count(d)
          plsc.addupdate_scatter(base_v, [d], occ + cnt_corr, mask=lastm)
          return 0
        lax.fori_loop(0, trips_s, shb, 0)

        def se(j, carry):
          g = base_v[pl.ds(j * 16, 16)]
          base_v[pl.ds(j * 16, 16)] = plsc.cumsum(g) - g + carry
          return carry + jnp.sum(g)
        lax.fori_loop(0, 16, se, jnp.int32(0))

        def ssc(i, _, _s_c=s_c, _s_p=s_p, _d_c=d_c, _d_p=d_p, _sh=sh):
          v = _s_c[pl.ds(i * 16, 16)]
          p = _s_p[pl.ds(i * 16, 16)]
          d = (lax.shift_right_logical(v, _sh) & 0xFF).astype(_i32)
          occ, lastm = plsc.scan_count(d)
          pos = plsc.load_gather(base_v, [d]) + (occ - base0)
          plsc.store_scatter(_d_c, [pos], v)
          plsc.store_scatter(_d_p, [pos], p)
          plsc.addupdate_scatter(base_v, [d], occ + cnt_corr, mask=lastm)
          return 0
        lax.fori_loop(0, trips_s, ssc, 0)

      toff = jnp.int32(0)
      woff = off
      for w in range(16):
        tw = cnt_smem[w, 1]
        sel = jnp.clip(rank - toff, 0, tw)
        for k in range(4):
          @pl.when(sel > 128 * k)
          def _(k=k, w=w):
            pltpu.sync_copy(sp_tie_p.at[w, pl.ds(128 * k, 128)],
                            asm_a.at[pl.ds(128 * k, 128)])
        def tb2(i, o, _sel=sel):
          m = (i * 16 + iota) < _sel
          plsc.store_compressed(uni_p.at[pl.ds(o, 16)],
                                asm_a[pl.ds(i * 16, 16)], mask=m)
          return o + pc(m)
        woff = lax.fori_loop(0, (sel + 15) >> 4, tb2, woff)
        toff = toff + tw

      def tf(i, _):
        lp = i * 16 + iota
        plsc.store_scatter(uni_c, [off + lp],
                           jnp.full((16,), 1, _i32) * t_acc, mask=lp < rank)
        return 0
      lax.fori_loop(0, (rank + 15) >> 4, tf, 0)

      def zf(j, _):
        fin_c[pl.ds(j * 16, 16)] = jnp.full((16,), -1, _i32)
        fin_p[pl.ds(j * 16, 16)] = jnp.zeros((16,), _i32)
        return 0
      lax.fori_loop(0, 32, zf, 0)

      def rp(i, _):
        r = i * 16 + iota
        q = r // NROW
        slot = r + q * (ROWS - NROW)
        plsc.store_scatter(fin_c, [slot], uni_c[pl.ds(i * 16, 16)])
        plsc.store_scatter(fin_p, [slot], uni_p[pl.ds(i * 16, 16)])
        return 0
      lax.fori_loop(0, KK // 16, rp, 0)

      pltpu.sync_copy(fin_c, sp_sorted_c)
      pltpu.sync_copy(fin_p, sp_sorted_p)
    plsc.subcore_barrier()

    pltpu.sync_copy(sp_sorted_c.at[pl.ds(ROWS * sidx, 32)], srt_c)
    pltpu.sync_copy(sp_sorted_p.at[pl.ds(ROWS * sidx, 32)], srt_p)

    for h in range(2):
      p = srt_p[pl.ds(h * 16, 16)]
      rows = lax.shift_right_logical(p, 4)
      idxg[pl.ds(h * 16, 16)] = b * NN + rows
      lb_st[pl.ds(h * 16, 16)] = p & 15
      v = srt_c[pl.ds(h * 16, 16)]
      sc_st[pl.ds(h * 16, 16)] = plsc.bitcast(~v, _f32)

    cp_ang = pltpu.async_copy(angles_hbm.at[idxg], angrows, sem_a)
    cp_box = pltpu.async_copy(boxes_hbm.at[idxg], boxrows, sem_b)

    pltpu.sync_copy(sc_st, scores_out.at[b, sidx])
    pltpu.sync_copy(lb_st, labels_out.at[b, sidx])

    cp_ang.wait()

    def am(r, _):
      def aj(j, carry):
        rmax, ridx = carry
        o = jnp.minimum(j * 16, 344)
        v = angrows[r, pl.ds(o, 16)]
        m = v > rmax
        return jnp.where(m, v, rmax), jnp.where(m, o + iota, ridx)
      rmax, ridx = lax.fori_loop(
          0, 23, aj, (jnp.full((16,), -jnp.inf, _f32), jnp.zeros((16,), _i32)))
      gmax = jnp.max(rmax)
      t0 = jnp.min(jnp.where(rmax == gmax, ridx, 100000))
      tv = t0 + iota * 90
      tv = jnp.where(tv >= 360, tv - 360, tv)
      plsc.store_scatter(th_st, [r * 4 + iota], tv, mask=iota < 4)
      return 0
    lax.fori_loop(0, NROW, am, 0)

    cp_box.wait()
    pltpu.sync_copy(th_st, theta_out.at[b, sidx])
    pltpu.sync_copy(boxrows, polys_out.at[b, sidx])
    return 0

  lax.fori_loop(0, 4, batch_body, 0)


@jax.jit
def _run_sc(c, boxes, angles, ts):
  f = pl.kernel(
      _sc_body,
      out_type=(
          jax.ShapeDtypeStruct((BB, NSUB, ROWS), _f32),
          jax.ShapeDtypeStruct((BB, NSUB, ROWS), _i32),
          jax.ShapeDtypeStruct((BB, NSUB, ROWS * 4), _i32),
          jax.ShapeDtypeStruct((BB, NSUB, ROWS, 32), _f32),
      ),
      mesh=plsc.VectorSubcoreMesh(core_axis_name="c", subcore_axis_name="s"),
      compiler_params=pltpu.CompilerParams(needs_layout_passes=False, use_tc_tiling_on_sc=False),
      scratch_types=[
          pltpu.VMEM((SHARD,), _i32),
          pltpu.VMEM((SHARD + 16,), _i32),
          pltpu.VMEM((SHARD + 16,), _i32),
          pltpu.VMEM((SHARD + 16,), _i32),
          pltpu.VMEM((SHARD + 16,), _i32),
          pltpu.VMEM((SHARD + 16,), _i32),
          pltpu.VMEM((512,), _i32),
          pltpu.VMEM((512,), _i32),
          pltpu.VMEM((256,), _i32),
          pltpu.VMEM((256,), _i32),
          pltpu.VMEM((512,), _i32),
          pltpu.VMEM((512,), _i32),
          pltpu.VMEM((512,), _i32),
          pltpu.VMEM((512,), _i32),
          pltpu.VMEM((256,), _i32),
          pltpu.VMEM((16,), _i32),
          pltpu.VMEM((16,), _f32),
          pltpu.VMEM((32,), _i32),
          pltpu.VMEM((32,), _i32),
          pltpu.VMEM((32,), _i32),
          pltpu.VMEM((ROWS, 32), _f32),
          pltpu.VMEM((ROWS, 360), _f32),
          pltpu.VMEM((32,), _f32),
          pltpu.VMEM((32,), _i32),
          pltpu.VMEM((128,), _i32),
          pltpu.SMEM((16, 16), _i32),
          pltpu.VMEM((512,), _i32),
          pltpu.VMEM((512,), _i32),
          pltpu.VMEM((512,), _i32),
          pltpu.VMEM((512,), _i32),
          pltpu.VMEM_SHARED((1024,), _i32),
          pltpu.VMEM_SHARED((16, 16), _i32),
          pltpu.VMEM_SHARED((16, 512), _i32),
          pltpu.VMEM_SHARED((16, 512), _i32),
          pltpu.VMEM_SHARED((16, 512), _i32),
          pltpu.VMEM_SHARED((512,), _i32),
          pltpu.VMEM_SHARED((512,), _i32),
          pltpu.SemaphoreType.DMA,
          pltpu.SemaphoreType.DMA,
      ],
  )
  return f(c, boxes, angles, ts)


def kernel(pred_logits, pred_boxes, pred_angles, target_sizes):
  prob = jax.nn.sigmoid(pred_logits)
  kbits = lax.bitcast_convert_type(prob, jnp.uint32)
  c = lax.bitcast_convert_type(~kbits, _i32)
  c = c.reshape(BB, NSUB, NPER)
  c = jnp.pad(c, ((0, 0), (0, 0), (0, SHARD - NPER)), constant_values=-1)
  boxes = jnp.pad(pred_boxes.reshape(BB * NN, 26), ((0, 0), (0, 6)))
  angles = pred_angles.reshape(BB * NN, 360)
  ts = target_sizes.reshape(16)

  scores_pad, labels_pad, theta_pad, polys_pad = _run_sc(c, boxes, angles, ts)

  scores = scores_pad[:, :, :NROW].reshape(BB, KK)
  labels = labels_pad[:, :, :NROW].reshape(BB, KK)
  theta = theta_pad[:, :, :NROW * 4].reshape(BB, KK, 4)
  img_w = target_sizes[:, 1]
  polys = (polys_pad[:, :, :NROW, :26].reshape(BB, KK, 13, 2)
           * img_w[:, None, None, None])
  valid = scores > 0.005
  return (scores, labels, polys, theta, valid)


# speedup vs baseline: 2.2139x; 1.8890x over previous
import functools

import jax
import jax.numpy as jnp
from jax import lax
from jax.experimental import pallas as pl
from jax.experimental.pallas import tpu as pltpu
from jax.experimental.pallas import tpu_sc as plsc

BB = 8
NN = 5000
CC = 16
KK = 400
NSUB = 16
NPER = NN * CC // NSUB
SHARD = 5120
NV = SHARD // 16
PADP = 0x7FFFFFFF
ROWS = 32
NROW = KK // NSUB

_i32 = jnp.int32
_f32 = jnp.float32


def _am_body(a_ref, o_ref):
  o_ref[...] = jnp.argmax(a_ref[...], axis=-1).astype(_i32)[:, None, :]


@jax.jit
def _tc_argmax(angles):
  return pl.pallas_call(
      _am_body,
      grid=(BB,),
      in_specs=[pl.BlockSpec((1, NN, 360), lambda i: (i, 0, 0))],
      out_specs=pl.BlockSpec((1, 1, NN), lambda i: (i, 0, 0)),
      out_shape=jax.ShapeDtypeStruct((BB, 1, NN), _i32),
  )(angles)


def _sc_body(c_hbm, boxes_hbm, th_hbm, ts_hbm,
             scores_out, labels_out, theta_out, polys_out,
             c_v, candA_c, candA_p, candB_c, candB_p, tie_p,
             above_c, above_p, hist_v, idx256, uni_c, uni_p, uni2_c, uni2_p,
             base_v, cntrow, ts_v, srt_c, srt_p, idxg, boxrows, throws,
             sc_st, lb_st, th_st, cnt_smem, asm_a, asm_b, fin_c, fin_p,
             sp_hist, sp_cnt, sp_above_c, sp_above_p, sp_tie_p,
             sp_sorted_c, sp_sorted_p, sem_a, sem_b):
  core = lax.axis_index("c")
  sidx = lax.axis_index("s")
  iota = lax.iota(_i32, 16)

  def pc(m):
    return jnp.sum(m.astype(_i32))

  occ0, _ = plsc.scan_count(jnp.zeros((16,), _i32))
  base0 = jnp.max(occ0) - 15
  cnt_corr = 1 - base0

  pltpu.sync_copy(ts_hbm, ts_v)

  def batch_body(bi, _):
    b = core * 4 + bi

    pltpu.sync_copy(c_hbm.at[pl.ds((b * NSUB + sidx) * SHARD, SHARD)], c_v)

    def zh(j, _):
      hist_v[pl.ds(j * 16, 16)] = jnp.zeros((16,), _i32)
      return 0
    lax.fori_loop(0, 16, zh, 0)

    @pl.when(sidx == 0)
    def _():
      for l in range(4):
        pltpu.sync_copy(hist_v, sp_hist.at[pl.ds(256 * l, 256)])
    plsc.subcore_barrier()

    rank = jnp.int32(KK)
    t_acc = jnp.int32(-2147483648)
    acnt = jnp.int32(0)
    ccnt = jnp.int32(0)
    shifts = [23, 15, 7, 0]
    masks = [0xFF, 0xFF, 0xFF, 0x7F]
    srcs = [(c_v, None), (candA_c, candA_p), (candB_c, candB_p),
            (candA_c, candA_p)]
    dsts = [(candA_c, candA_p), (candB_c, candB_p), (candA_c, candA_p),
            (candB_c, candB_p)]

    for l in range(4):
      sh, mk = shifts[l], masks[l]
      src_c, src_p = srcs[l]
      dst_c, dst_p = dsts[l]
      trips = NV if l == 0 else (ccnt + 15) >> 4

      def dig(v):
        return (lax.shift_right_logical(v, sh) & mk).astype(_i32)

      def zh2(j, _):
        hist_v[pl.ds(j * 16, 16)] = jnp.zeros((16,), _i32)
        return 0
      lax.fori_loop(0, 16, zh2, 0)

      if l == 0:
        def hb(i, _):
          v = src_c[pl.ds(i * 16, 16)]
          d = dig(v)
          occ, lastm = plsc.scan_count(d)
          plsc.addupdate_scatter(hist_v, [d], occ + cnt_corr, mask=lastm)
          return 0
      else:
        def hb(i, _, _cc=ccnt, _src=src_c, _dig=dig):
          v = _src[pl.ds(i * 16, 16)]
          valid = (i * 16 + iota) < _cc
          d = _dig(v)
          occ, lastm = plsc.scan_count(d, mask=valid)
          plsc.addupdate_scatter(hist_v, [d], occ + cnt_corr, mask=lastm)
          return 0
      lax.fori_loop(0, trips, hb, 0)

      def wi(j, _):
        idx256[pl.ds(j * 16, 16)] = 256 * l + j * 16 + iota
        return 0
      lax.fori_loop(0, 16, wi, 0)
      pltpu.sync_copy(hist_v, sp_hist.at[idx256], add=True)
      plsc.subcore_barrier()
      pltpu.sync_copy(sp_hist.at[pl.ds(256 * l, 256)], hist_v)

      def sb(j, carry):
        run, nlt, abv = carry
        g = hist_v[pl.ds(j * 16, 16)]
        cum = plsc.cumsum(g) + run
        m = cum < rank
        return (run + jnp.sum(g), nlt + pc(m),
                jnp.maximum(abv, jnp.max(jnp.where(m, cum, 0))))
      _, bl, abv = lax.fori_loop(0, 16, sb, (jnp.int32(0), jnp.int32(0),
                                             jnp.int32(0)))
      rank = rank - abv
      t_acc = t_acc | lax.shift_left(bl, sh)

      if l == 0:
        def cb(i, carry, _bl=bl, _dig=dig):
          ac, cc = carry
          v = c_v[pl.ds(i * 16, 16)]
          lp = i * 16 + iota
          p = jnp.where(lp < NPER, sidx * NPER + lp, PADP)
          d = _dig(v)
          m_ab = d < _bl
          m_in = d == _bl
          plsc.store_compressed(above_c.at[pl.ds(ac, 16)], v, mask=m_ab)
          plsc.store_compressed(above_p.at[pl.ds(ac, 16)], p, mask=m_ab)
          plsc.store_compressed(candA_c.at[pl.ds(cc, 16)], v, mask=m_in)
          plsc.store_compressed(candA_p.at[pl.ds(cc, 16)], p, mask=m_in)
          return ac + pc(m_ab), cc + pc(m_in)
      else:
        def cb(i, carry, _bl=bl, _cc=ccnt, _src_c=src_c, _src_p=src_p,
               _dst_c=dst_c, _dst_p=dst_p, _dig=dig):
          ac, cc = carry
          v = _src_c[pl.ds(i * 16, 16)]
          p = _src_p[pl.ds(i * 16, 16)]
          valid = (i * 16 + iota) < _cc
          d = _dig(v)
          m_ab = (d < _bl) & valid
          m_in = (d == _bl) & valid
          plsc.store_compressed(above_c.at[pl.ds(ac, 16)], v, mask=m_ab)
          plsc.store_compressed(above_p.at[pl.ds(ac, 16)], p, mask=m_ab)
          plsc.store_compressed(_dst_c.at[pl.ds(cc, 16)], v, mask=m_in)
          plsc.store_compressed(_dst_p.at[pl.ds(cc, 16)], p, mask=m_in)
          return ac + pc(m_ab), cc + pc(m_in)
      acnt, ccnt = lax.fori_loop(0, trips, cb, (acnt, jnp.int32(0)))

    def tb(i, tc):
      p = candB_p[pl.ds(i * 16, 16)]
      m = ((i * 16 + iota) < ccnt) & (p != PADP)
      plsc.store_compressed(tie_p.at[pl.ds(tc, 16)], p, mask=m)
      return tc + pc(m)
    tcnt = lax.fori_loop(0, (ccnt + 15) >> 4, tb, jnp.int32(0))
    tpub = jnp.minimum(tcnt, 512)

    cntrow[...] = jnp.where(iota == 0, acnt, jnp.where(iota == 1, tpub, 0))
    pltpu.sync_copy(cntrow, sp_cnt.at[sidx])
    for k in range(4):
      @pl.when(acnt > 128 * k)
      def _(k=k):
        pltpu.sync_copy(above_c.at[pl.ds(128 * k, 128)],
                        sp_above_c.at[sidx, pl.ds(128 * k, 128)])
        pltpu.sync_copy(above_p.at[pl.ds(128 * k, 128)],
                        sp_above_p.at[sidx, pl.ds(128 * k, 128)])
      @pl.when(tpub > 128 * k)
      def _(k=k):
        pltpu.sync_copy(tie_p.at[pl.ds(128 * k, 128)],
                        sp_tie_p.at[sidx, pl.ds(128 * k, 128)])
    plsc.subcore_barrier()

    @pl.when(sidx == 0)
    def _():
      pltpu.sync_copy(sp_cnt, cnt_smem)
      def zi(j, _):
        uni_c[pl.ds(j * 16, 16)] = jnp.full((16,), -1, _i32)
        uni_p[pl.ds(j * 16, 16)] = jnp.zeros((16,), _i32)
        uni2_c[pl.ds(j * 16, 16)] = jnp.full((16,), -1, _i32)
        uni2_p[pl.ds(j * 16, 16)] = jnp.zeros((16,), _i32)
        return 0
      lax.fori_loop(0, 32, zi, 0)

      off = jnp.int32(0)
      for w in range(16):
        ca = cnt_smem[w, 0]
        for k in range(4):
          @pl.when(ca > 128 * k)
          def _(k=k, w=w):
            pltpu.sync_copy(sp_above_c.at[w, pl.ds(128 * k, 128)],
                            asm_a.at[pl.ds(128 * k, 128)])
            pltpu.sync_copy(sp_above_p.at[w, pl.ds(128 * k, 128)],
                            asm_b.at[pl.ds(128 * k, 128)])
        def ab(i, o, _ca=ca):
          m = (i * 16 + iota) < _ca
          plsc.store_compressed(uni_c.at[pl.ds(o, 16)],
                                asm_a[pl.ds(i * 16, 16)], mask=m)
          plsc.store_compressed(uni_p.at[pl.ds(o, 16)],
                                asm_b[pl.ds(i * 16, 16)], mask=m)
          return o + pc(m)
        off = lax.fori_loop(0, (ca + 15) >> 4, ab, off)

      trips_s = (off + 15) >> 4
      bufs = [(uni_c, uni_p, uni2_c, uni2_p), (uni2_c, uni2_p, uni_c, uni_p)]
      for pi, sh in enumerate([0, 8, 16, 24]):
        s_c, s_p, d_c, d_p = bufs[pi % 2]

        def zb(j, _):
          base_v[pl.ds(j * 16, 16)] = jnp.zeros((16,), _i32)
          return 0
        lax.fori_loop(0, 16, zb, 0)

        def shb(i, _, _s_c=s_c, _sh=sh):
          v = _s_c[pl.ds(i * 16, 16)]
          d = (lax.shift_right_logical(v, _sh) & 0xFF).astype(_i32)
          occ, lastm = plsc.scan_count(d)
          plsc.addupdate_scatter(base_v, [d], occ + cnt_corr, mask=lastm)
          return 0
        lax.fori_loop(0, trips_s, shb, 0)

        def se(j, carry):
          g = base_v[pl.ds(j * 16, 16)]
          base_v[pl.ds(j * 16, 16)] = plsc.cumsum(g) - g + carry
          return carry + jnp.sum(g)
        lax.fori_loop(0, 16, se, jnp.int32(0))

        def ssc(i, _, _s_c=s_c, _s_p=s_p, _d_c=d_c, _d_p=d_p, _sh=sh):
          v = _s_c[pl.ds(i * 16, 16)]
          p = _s_p[pl.ds(i * 16, 16)]
          d = (lax.shift_right_logical(v, _sh) & 0xFF).astype(_i32)
          occ, lastm = plsc.scan_count(d)
          pos = plsc.load_gather(base_v, [d]) + (occ - base0)
          plsc.store_scatter(_d_c, [pos], v)
          plsc.store_scatter(_d_p, [pos], p)
          plsc.addupdate_scatter(base_v, [d], occ + cnt_corr, mask=lastm)
          return 0
        lax.fori_loop(0, trips_s, ssc, 0)

      toff = jnp.int32(0)
      woff = off
      for w in range(16):
        tw = cnt_smem[w, 1]
        sel = jnp.clip(rank - toff, 0, tw)
        for k in range(4):
          @pl.when(sel > 128 * k)
          def _(k=k, w=w):
            pltpu.sync_copy(sp_tie_p.at[w, pl.ds(128 * k, 128)],
                            asm_a.at[pl.ds(128 * k, 128)])
        def tb2(i, o, _sel=sel):
          m = (i * 16 + iota) < _sel
          plsc.store_compressed(uni_p.at[pl.ds(o, 16)],
                                asm_a[pl.ds(i * 16, 16)], mask=m)
          return o + pc(m)
        woff = lax.fori_loop(0, (sel + 15) >> 4, tb2, woff)
        toff = toff + tw

      def tf(i, _):
        lp = i * 16 + iota
        plsc.store_scatter(uni_c, [off + lp],
                           jnp.full((16,), 1, _i32) * t_acc, mask=lp < rank)
        return 0
      lax.fori_loop(0, (rank + 15) >> 4, tf, 0)

      def zf(j, _):
        fin_c[pl.ds(j * 16, 16)] = jnp.full((16,), -1, _i32)
        fin_p[pl.ds(j * 16, 16)] = jnp.zeros((16,), _i32)
        return 0
      lax.fori_loop(0, 32, zf, 0)

      def rp(i, _):
        r = i * 16 + iota
        q = r // NROW
        slot = r + q * (ROWS - NROW)
        plsc.store_scatter(fin_c, [slot], uni_c[pl.ds(i * 16, 16)])
        plsc.store_scatter(fin_p, [slot], uni_p[pl.ds(i * 16, 16)])
        return 0
      lax.fori_loop(0, KK // 16, rp, 0)

      pltpu.sync_copy(fin_c, sp_sorted_c)
      pltpu.sync_copy(fin_p, sp_sorted_p)
    plsc.subcore_barrier()

    pltpu.sync_copy(sp_sorted_c.at[pl.ds(ROWS * sidx, 32)], srt_c)
    pltpu.sync_copy(sp_sorted_p.at[pl.ds(ROWS * sidx, 32)], srt_p)

    for h in range(2):
      p = srt_p[pl.ds(h * 16, 16)]
      rows = lax.shift_right_logical(p, 4)
      idxg[pl.ds(h * 16, 16)] = b * NN + rows
      lb_st[pl.ds(h * 16, 16)] = p & 15
      v = srt_c[pl.ds(h * 16, 16)]
      sc_st[pl.ds(h * 16, 16)] = plsc.bitcast(~v, _f32)

    cp_th = pltpu.async_copy(th_hbm.at[idxg], throws, sem_a)
    cp_box = pltpu.async_copy(boxes_hbm.at[idxg], boxrows, sem_b)

    pltpu.sync_copy(sc_st, scores_out.at[b, sidx])
    pltpu.sync_copy(lb_st, labels_out.at[b, sidx])

    cp_th.wait()
    for h in range(2):
      t0 = throws[pl.ds(h * 16, 16)]
      rowid = h * 16 + iota
      for rot in range(4):
        tv = t0 + 90 * rot
        tv = jnp.where(tv >= 360, tv - 360, tv)
        plsc.store_scatter(th_st, [rowid * 4 + rot], tv)
    pltpu.sync_copy(th_st, theta_out.at[b, sidx])

    cp_box.wait()
    pltpu.sync_copy(boxrows, polys_out.at[b, sidx])
    return 0

  lax.fori_loop(0, 4, batch_body, 0)


@jax.jit
def _run_sc(c, boxes, th, ts):
  f = pl.kernel(
      _sc_body,
      out_type=(
          jax.ShapeDtypeStruct((BB, NSUB, ROWS), _f32),
          jax.ShapeDtypeStruct((BB, NSUB, ROWS), _i32),
          jax.ShapeDtypeStruct((BB, NSUB, ROWS * 4), _i32),
          jax.ShapeDtypeStruct((BB, NSUB, ROWS, 32), _f32),
      ),
      mesh=plsc.VectorSubcoreMesh(core_axis_name="c", subcore_axis_name="s"),
      compiler_params=pltpu.CompilerParams(needs_layout_passes=False, use_tc_tiling_on_sc=False),
      scratch_types=[
          pltpu.VMEM((SHARD,), _i32),
          pltpu.VMEM((SHARD + 16,), _i32),
          pltpu.VMEM((SHARD + 16,), _i32),
          pltpu.VMEM((SHARD + 16,), _i32),
          pltpu.VMEM((SHARD + 16,), _i32),
          pltpu.VMEM((SHARD + 16,), _i32),
          pltpu.VMEM((512,), _i32),
          pltpu.VMEM((512,), _i32),
          pltpu.VMEM((256,), _i32),
          pltpu.VMEM((256,), _i32),
          pltpu.VMEM((512,), _i32),
          pltpu.VMEM((512,), _i32),
          pltpu.VMEM((512,), _i32),
          pltpu.VMEM((512,), _i32),
          pltpu.VMEM((256,), _i32),
          pltpu.VMEM((16,), _i32),
          pltpu.VMEM((16,), _f32),
          pltpu.VMEM((32,), _i32),
          pltpu.VMEM((32,), _i32),
          pltpu.VMEM((32,), _i32),
          pltpu.VMEM((ROWS, 32), _f32),
          pltpu.VMEM((32,), _i32),
          pltpu.VMEM((32,), _f32),
          pltpu.VMEM((32,), _i32),
          pltpu.VMEM((128,), _i32),
          pltpu.SMEM((16, 16), _i32),
          pltpu.VMEM((512,), _i32),
          pltpu.VMEM((512,), _i32),
          pltpu.VMEM((512,), _i32),
          pltpu.VMEM((512,), _i32),
          pltpu.VMEM_SHARED((1024,), _i32),
          pltpu.VMEM_SHARED((16, 16), _i32),
          pltpu.VMEM_SHARED((16, 512), _i32),
          pltpu.VMEM_SHARED((16, 512), _i32),
          pltpu.VMEM_SHARED((16, 512), _i32),
          pltpu.VMEM_SHARED((512,), _i32),
          pltpu.VMEM_SHARED((512,), _i32),
          pltpu.SemaphoreType.DMA,
          pltpu.SemaphoreType.DMA,
      ],
  )
  return f(c, boxes, th, ts)


def kernel(pred_logits, pred_boxes, pred_angles, target_sizes):
  prob = jax.nn.sigmoid(pred_logits)
  kbits = lax.bitcast_convert_type(prob, jnp.uint32)
  c = lax.bitcast_convert_type(~kbits, _i32)
  c = c.reshape(BB, NSUB, NPER)
  c = jnp.pad(c, ((0, 0), (0, 0), (0, SHARD - NPER)), constant_values=-1)
  c = c.reshape(BB * NSUB * SHARD)
  boxes = jnp.pad(pred_boxes.reshape(BB * NN, 26), ((0, 0), (0, 6)))
  th = _tc_argmax(pred_angles).reshape(BB * NN)
  ts = target_sizes.reshape(16)

  scores_pad, labels_pad, theta_pad, polys_pad = _run_sc(c, boxes, th, ts)

  scores = scores_pad[:, :, :NROW].reshape(BB, KK)
  labels = labels_pad[:, :, :NROW].reshape(BB, KK)
  theta = theta_pad[:, :, :NROW * 4].reshape(BB, KK, 4)
  img_w = target_sizes[:, 1]
  polys = (polys_pad[:, :, :NROW, :26].reshape(BB, KK, 13, 2)
           * img_w[:, None, None, None])
  valid = scores > 0.005
  return (scores, labels, polys, theta, valid)


# speedup vs baseline: 2.4425x; 1.1033x over previous
import functools

import jax
import jax.numpy as jnp
from jax import lax
from jax.experimental import pallas as pl
from jax.experimental.pallas import tpu as pltpu
from jax.experimental.pallas import tpu_sc as plsc

BB = 8
NN = 5000
CC = 16
KK = 400
NSUB = 16
NPER = NN * CC // NSUB
SHARD = 5120
NV = SHARD // 16
PADP = 0x7FFFFFFF
ROWS = 32
NROW = KK // NSUB

_i32 = jnp.int32
_f32 = jnp.float32


def _am_body(a_ref, o_ref):
  o_ref[...] = jnp.argmax(a_ref[...], axis=-1).astype(_i32)[:, None, :]


@jax.jit
def _tc_argmax(angles):
  return pl.pallas_call(
      _am_body,
      grid=(BB,),
      in_specs=[pl.BlockSpec((1, NN, 360), lambda i: (i, 0, 0))],
      out_specs=pl.BlockSpec((1, 1, NN), lambda i: (i, 0, 0)),
      out_shape=jax.ShapeDtypeStruct((BB, 1, NN), _i32),
  )(angles)


def _sc_body(c_hbm, boxes_hbm, th_hbm, ts_hbm,
             scores_out, labels_out, theta_out, polys_out,
             c_v, candA_c, candA_p, candB_c, candB_p, tie_p,
             above_c, above_p, hist_v, idx256, uni_c, uni_p, uni2_c, uni2_p,
             base_v, cntrow, ts_v, srt_c, srt_p, idxg, boxrows, throws,
             sc_st, lb_st, th_st, cnt_smem, asm_a, asm_b, fin_c, fin_p, zro_v,
             sp_hist, sp_cnt, sp_above_c, sp_above_p, sp_tie_p,
             sp_sorted_c, sp_sorted_p, sem_a, sem_b):
  core = lax.axis_index("c")
  sidx = lax.axis_index("s")
  iota = lax.iota(_i32, 16)

  def pc(m):
    return jnp.sum(m.astype(_i32))

  occ0, _ = plsc.scan_count(jnp.zeros((16,), _i32))
  base0 = jnp.max(occ0) - 15
  cnt_corr = 1 - base0

  pltpu.sync_copy(ts_hbm, ts_v)

  def zz(j, _):
    zro_v[pl.ds(j * 16, 16)] = jnp.zeros((16,), _i32)
    return 0
  lax.fori_loop(0, 16, zz, 0)

  @pl.when(sidx == 0)
  def _():
    for l in range(4):
      pltpu.sync_copy(zro_v, sp_hist.at[pl.ds(256 * l, 256)])
  plsc.subcore_barrier()

  def batch_body(bi, _):
    b = core * 4 + bi

    pltpu.sync_copy(c_hbm.at[pl.ds((b * NSUB + sidx) * SHARD, SHARD)], c_v)

    rank = jnp.int32(KK)
    t_acc = jnp.int32(-2147483648)
    acnt = jnp.int32(0)
    ccnt = jnp.int32(0)
    shifts = [23, 15, 7, 0]
    masks = [0xFF, 0xFF, 0xFF, 0x7F]
    srcs = [(c_v, None), (candA_c, candA_p), (candB_c, candB_p),
            (candA_c, candA_p)]
    dsts = [(candA_c, candA_p), (candB_c, candB_p), (candA_c, candA_p),
            (candB_c, candB_p)]

    for l in range(4):
      sh, mk = shifts[l], masks[l]
      src_c, src_p = srcs[l]
      dst_c, dst_p = dsts[l]
      trips = NV if l == 0 else (ccnt + 15) >> 4

      def dig(v):
        return (lax.shift_right_logical(v, sh) & mk).astype(_i32)

      def zh2(j, _):
        hist_v[pl.ds(j * 16, 16)] = jnp.zeros((16,), _i32)
        return 0
      lax.fori_loop(0, 16, zh2, 0)

      if l == 0:
        @plsc.parallel_loop(0, NV, step=1, unroll=8)
        def _(i):
          v = src_c[pl.ds(i * 16, 16)]
          d = dig(v)
          occ, lastm = plsc.scan_count(d)
          plsc.addupdate_scatter(hist_v, [d], occ + cnt_corr, mask=lastm)
      else:
        def hb(i, _, _cc=ccnt, _src=src_c, _dig=dig):
          v = _src[pl.ds(i * 16, 16)]
          valid = (i * 16 + iota) < _cc
          d = _dig(v)
          occ, lastm = plsc.scan_count(d, mask=valid)
          plsc.addupdate_scatter(hist_v, [d], occ + cnt_corr, mask=lastm)
          return 0
        lax.fori_loop(0, trips, hb, 0)

      def wi(j, _):
        idx256[pl.ds(j * 16, 16)] = 256 * l + j * 16 + iota
        return 0
      lax.fori_loop(0, 16, wi, 0)
      pltpu.sync_copy(hist_v, sp_hist.at[idx256], add=True)
      plsc.subcore_barrier()
      if l > 0:
        @pl.when(sidx == 0)
        def _(l=l):
          pltpu.sync_copy(zro_v, sp_hist.at[pl.ds(256 * (l - 1), 256)])
      pltpu.sync_copy(sp_hist.at[pl.ds(256 * l, 256)], hist_v)

      def sb(j, carry):
        run, nlt, abv = carry
        g = hist_v[pl.ds(j * 16, 16)]
        cum = plsc.cumsum(g) + run
        m = cum < rank
        return (run + jnp.sum(g), nlt + pc(m),
                jnp.maximum(abv, jnp.max(jnp.where(m, cum, 0))))
      _, bl, abv = lax.fori_loop(0, 16, sb, (jnp.int32(0), jnp.int32(0),
                                             jnp.int32(0)))
      rank = rank - abv
      t_acc = t_acc | lax.shift_left(bl, sh)

      if l == 0:
        @plsc.parallel_loop(0, NV, step=1, unroll=4,
                            carry=(acnt, jnp.int32(0)))
        def _cl(i, carry, _bl=bl, _dig=dig):
          ac, cc = carry
          v = c_v[pl.ds(i * 16, 16)]
          lp = i * 16 + iota
          p = jnp.where(lp < NPER, sidx * NPER + lp, PADP)
          d = _dig(v)
          m_ab = d < _bl
          m_in = d == _bl
          plsc.store_compressed(above_c.at[pl.ds(ac, 16)], v, mask=m_ab)
          plsc.store_compressed(above_p.at[pl.ds(ac, 16)], p, mask=m_ab)
          plsc.store_compressed(candA_c.at[pl.ds(cc, 16)], v, mask=m_in)
          plsc.store_compressed(candA_p.at[pl.ds(cc, 16)], p, mask=m_in)
          return ac + pc(m_ab), cc + pc(m_in)
        acnt, ccnt = _cl
      else:
        def cb(i, carry, _bl=bl, _cc=ccnt, _src_c=src_c, _src_p=src_p,
               _dst_c=dst_c, _dst_p=dst_p, _dig=dig):
          ac, cc = carry
          v = _src_c[pl.ds(i * 16, 16)]
          p = _src_p[pl.ds(i * 16, 16)]
          valid = (i * 16 + iota) < _cc
          d = _dig(v)
          m_ab = (d < _bl) & valid
          m_in = (d == _bl) & valid
          plsc.store_compressed(above_c.at[pl.ds(ac, 16)], v, mask=m_ab)
          plsc.store_compressed(above_p.at[pl.ds(ac, 16)], p, mask=m_ab)
          plsc.store_compressed(_dst_c.at[pl.ds(cc, 16)], v, mask=m_in)
          plsc.store_compressed(_dst_p.at[pl.ds(cc, 16)], p, mask=m_in)
          return ac + pc(m_ab), cc + pc(m_in)
        acnt, ccnt = lax.fori_loop(0, trips, cb, (acnt, jnp.int32(0)))

    def tb(i, tc):
      p = candB_p[pl.ds(i * 16, 16)]
      m = ((i * 16 + iota) < ccnt) & (p != PADP)
      plsc.store_compressed(tie_p.at[pl.ds(tc, 16)], p, mask=m)
      return tc + pc(m)
    tcnt = lax.fori_loop(0, (ccnt + 15) >> 4, tb, jnp.int32(0))
    tpub = jnp.minimum(tcnt, 512)

    cntrow[...] = jnp.where(iota == 0, acnt, jnp.where(iota == 1, tpub, 0))
    pltpu.sync_copy(cntrow, sp_cnt.at[sidx])
    for k in range(4):
      @pl.when(acnt > 128 * k)
      def _(k=k):
        pltpu.sync_copy(above_c.at[pl.ds(128 * k, 128)],
                        sp_above_c.at[sidx, pl.ds(128 * k, 128)])
        pltpu.sync_copy(above_p.at[pl.ds(128 * k, 128)],
                        sp_above_p.at[sidx, pl.ds(128 * k, 128)])
      @pl.when(tpub > 128 * k)
      def _(k=k):
        pltpu.sync_copy(tie_p.at[pl.ds(128 * k, 128)],
                        sp_tie_p.at[sidx, pl.ds(128 * k, 128)])
    plsc.subcore_barrier()

    @pl.when(sidx == 0)
    def _():
      pltpu.sync_copy(zro_v, sp_hist.at[pl.ds(256 * 3, 256)])
      pltpu.sync_copy(sp_cnt, cnt_smem)
      def zi(j, _):
        uni_c[pl.ds(j * 16, 16)] = jnp.full((16,), -1, _i32)
        uni_p[pl.ds(j * 16, 16)] = jnp.zeros((16,), _i32)
        uni2_c[pl.ds(j * 16, 16)] = jnp.full((16,), -1, _i32)
        uni2_p[pl.ds(j * 16, 16)] = jnp.zeros((16,), _i32)
        return 0
      lax.fori_loop(0, 32, zi, 0)

      off = jnp.int32(0)
      for w in range(16):
        ca = cnt_smem[w, 0]
        for k in range(4):
          @pl.when(ca > 128 * k)
          def _(k=k, w=w):
            pltpu.sync_copy(sp_above_c.at[w, pl.ds(128 * k, 128)],
                            asm_a.at[pl.ds(128 * k, 128)])
            pltpu.sync_copy(sp_above_p.at[w, pl.ds(128 * k, 128)],
                            asm_b.at[pl.ds(128 * k, 128)])
        def ab(i, o, _ca=ca):
          m = (i * 16 + iota) < _ca
          plsc.store_compressed(uni_c.at[pl.ds(o, 16)],
                                asm_a[pl.ds(i * 16, 16)], mask=m)
          plsc.store_compressed(uni_p.at[pl.ds(o, 16)],
                                asm_b[pl.ds(i * 16, 16)], mask=m)
          return o + pc(m)
        off = lax.fori_loop(0, (ca + 15) >> 4, ab, off)

      trips_s = (off + 15) >> 4
      bufs = [(uni_c, uni_p, uni2_c, uni2_p), (uni2_c, uni2_p, uni_c, uni_p)]
      for pi, sh in enumerate([0, 8, 16, 24]):
        s_c, s_p, d_c, d_p = bufs[pi % 2]

        def zb(j, _):
          base_v[pl.ds(j * 16, 16)] = jnp.zeros((16,), _i32)
          return 0
        lax.fori_loop(0, 16, zb, 0)

        def shb(i, _, _s_c=s_c, _sh=sh):
          v = _s_c[pl.ds(i * 16, 16)]
          d = (lax.shift_right_logical(v, _sh) & 0xFF).astype(_i32)
          occ, lastm = plsc.scan_count(d)
          plsc.addupdate_scatter(base_v, [d], occ + cnt_corr, mask=lastm)
          return 0
        lax.fori_loop(0, trips_s, shb, 0)

        def se(j, carry):
          g = base_v[pl.ds(j * 16, 16)]
          base_v[pl.ds(j * 16, 16)] = plsc.cumsum(g) - g + carry
          return carry + jnp.sum(g)
        lax.fori_loop(0, 16, se, jnp.int32(0))

        def ssc(i, _, _s_c=s_c, _s_p=s_p, _d_c=d_c, _d_p=d_p, _sh=sh):
          v = _s_c[pl.ds(i * 16, 16)]
          p = _s_p[pl.ds(i * 16, 16)]
          d = (lax.shift_right_logical(v, _sh) & 0xFF).astype(_i32)
          occ, lastm = plsc.scan_count(d)
          pos = plsc.load_gather(base_v, [d]) + (occ - base0)
          plsc.store_scatter(_d_c, [pos], v)
          plsc.store_scatter(_d_p, [pos], p)
          plsc.addupdate_scatter(base_v, [d], occ + cnt_corr, mask=lastm)
          return 0
        lax.fori_loop(0, trips_s, ssc, 0)

      toff = jnp.int32(0)
      woff = off
      for w in range(16):
        tw = cnt_smem[w, 1]
        sel = jnp.clip(rank - toff, 0, tw)
        for k in range(4):
          @pl.when(sel > 128 * k)
          def _(k=k, w=w):
            pltpu.sync_copy(sp_tie_p.at[w, pl.ds(128 * k, 128)],
                            asm_a.at[pl.ds(128 * k, 128)])
        def tb2(i, o, _sel=sel):
          m = (i * 16 + iota) < _sel
          plsc.store_compressed(uni_p.at[pl.ds(o, 16)],
                                asm_a[pl.ds(i * 16, 16)], mask=m)
          return o + pc(m)
        woff = lax.fori_loop(0, (sel + 15) >> 4, tb2, woff)
        toff = toff + tw

      def tf(i, _):
        lp = i * 16 + iota
        plsc.store_scatter(uni_c, [off + lp],
                           jnp.full((16,), 1, _i32) * t_acc, mask=lp < rank)
        return 0
      lax.fori_loop(0, (rank + 15) >> 4, tf, 0)

      def zf(j, _):
        fin_c[pl.ds(j * 16, 16)] = jnp.full((16,), -1, _i32)
        fin_p[pl.ds(j * 16, 16)] = jnp.zeros((16,), _i32)
        return 0
      lax.fori_loop(0, 32, zf, 0)

      def rp(i, _):
        r = i * 16 + iota
        q = r // NROW
        slot = r + q * (ROWS - NROW)
        plsc.store_scatter(fin_c, [slot], uni_c[pl.ds(i * 16, 16)])
        plsc.store_scatter(fin_p, [slot], uni_p[pl.ds(i * 16, 16)])
        return 0
      lax.fori_loop(0, KK // 16, rp, 0)

      pltpu.sync_copy(fin_c, sp_sorted_c)
      pltpu.sync_copy(fin_p, sp_sorted_p)
    plsc.subcore_barrier()

    pltpu.sync_copy(sp_sorted_c.at[pl.ds(ROWS * sidx, 32)], srt_c)
    pltpu.sync_copy(sp_sorted_p.at[pl.ds(ROWS * sidx, 32)], srt_p)

    for h in range(2):
      p = srt_p[pl.ds(h * 16, 16)]
      rows = lax.shift_right_logical(p, 4)
      idxg[pl.ds(h * 16, 16)] = b * NN + rows
      lb_st[pl.ds(h * 16, 16)] = p & 15
      v = srt_c[pl.ds(h * 16, 16)]
      sc_st[pl.ds(h * 16, 16)] = plsc.bitcast(~v, _f32)

    cp_th = pltpu.async_copy(th_hbm.at[idxg], throws, sem_a)
    cp_box = pltpu.async_copy(boxes_hbm.at[idxg], boxrows, sem_b)

    pltpu.sync_copy(sc_st, scores_out.at[b, sidx])
    pltpu.sync_copy(lb_st, labels_out.at[b, sidx])

    cp_th.wait()
    for h in range(2):
      t0 = throws[pl.ds(h * 16, 16)]
      rowid = h * 16 + iota
      for rot in range(4):
        tv = t0 + 90 * rot
        tv = jnp.where(tv >= 360, tv - 360, tv)
        plsc.store_scatter(th_st, [rowid * 4 + rot], tv)
    pltpu.sync_copy(th_st, theta_out.at[b, sidx])

    cp_box.wait()
    pltpu.sync_copy(boxrows, polys_out.at[b, sidx])
    return 0

  lax.fori_loop(0, 4, batch_body, 0)


@jax.jit
def _run_sc(c, boxes, th, ts):
  f = pl.kernel(
      _sc_body,
      out_type=(
          jax.ShapeDtypeStruct((BB, NSUB, ROWS), _f32),
          jax.ShapeDtypeStruct((BB, NSUB, ROWS), _i32),
          jax.ShapeDtypeStruct((BB, NSUB, ROWS * 4), _i32),
          jax.ShapeDtypeStruct((BB, NSUB, ROWS, 32), _f32),
      ),
      mesh=plsc.VectorSubcoreMesh(core_axis_name="c", subcore_axis_name="s"),
      compiler_params=pltpu.CompilerParams(needs_layout_passes=False, use_tc_tiling_on_sc=False),
      scratch_types=[
          pltpu.VMEM((SHARD,), _i32),
          pltpu.VMEM((SHARD + 16,), _i32),
          pltpu.VMEM((SHARD + 16,), _i32),
          pltpu.VMEM((SHARD + 16,), _i32),
          pltpu.VMEM((SHARD + 16,), _i32),
          pltpu.VMEM((SHARD + 16,), _i32),
          pltpu.VMEM((512,), _i32),
          pltpu.VMEM((512,), _i32),
          pltpu.VMEM((256,), _i32),
          pltpu.VMEM((256,), _i32),
          pltpu.VMEM((512,), _i32),
          pltpu.VMEM((512,), _i32),
          pltpu.VMEM((512,), _i32),
          pltpu.VMEM((512,), _i32),
          pltpu.VMEM((256,), _i32),
          pltpu.VMEM((16,), _i32),
          pltpu.VMEM((16,), _f32),
          pltpu.VMEM((32,), _i32),
          pltpu.VMEM((32,), _i32),
          pltpu.VMEM((32,), _i32),
          pltpu.VMEM((ROWS, 32), _f32),
          pltpu.VMEM((32,), _i32),
          pltpu.VMEM((32,), _f32),
          pltpu.VMEM((32,), _i32),
          pltpu.VMEM((128,), _i32),
          pltpu.SMEM((16, 16), _i32),
          pltpu.VMEM((512,), _i32),
          pltpu.VMEM((512,), _i32),
          pltpu.VMEM((512,), _i32),
          pltpu.VMEM((512,), _i32),
          pltpu.VMEM((256,), _i32),
          pltpu.VMEM_SHARED((1024,), _i32),
          pltpu.VMEM_SHARED((16, 16), _i32),
          pltpu.VMEM_SHARED((16, 512), _i32),
          pltpu.VMEM_SHARED((16, 512), _i32),
          pltpu.VMEM_SHARED((16, 512), _i32),
          pltpu.VMEM_SHARED((512,), _i32),
          pltpu.VMEM_SHARED((512,), _i32),
          pltpu.SemaphoreType.DMA,
          pltpu.SemaphoreType.DMA,
      ],
  )
  return f(c, boxes, th, ts)


def kernel(pred_logits, pred_boxes, pred_angles, target_sizes):
  prob = jax.nn.sigmoid(pred_logits)
  kbits = lax.bitcast_convert_type(prob, jnp.uint32)
  c = lax.bitcast_convert_type(~kbits, _i32)
  c = c.reshape(BB, NSUB, NPER)
  c = jnp.pad(c, ((0, 0), (0, 0), (0, SHARD - NPER)), constant_values=-1)
  c = c.reshape(BB * NSUB * SHARD)
  boxes = jnp.pad(pred_boxes.reshape(BB * NN, 26), ((0, 0), (0, 6)))
  th = _tc_argmax(pred_angles).reshape(BB * NN)
  ts = target_sizes.reshape(16)

  scores_pad, labels_pad, theta_pad, polys_pad = _run_sc(c, boxes, th, ts)

  scores = scores_pad[:, :, :NROW].reshape(BB, KK)
  labels = labels_pad[:, :, :NROW].reshape(BB, KK)
  theta = theta_pad[:, :, :NROW * 4].reshape(BB, KK, 4)
  img_w = target_sizes[:, 1]
  polys = (polys_pad[:, :, :NROW, :26].reshape(BB, KK, 13, 2)
           * img_w[:, None, None, None])
  valid = scores > 0.005
  return (scores, labels, polys, theta, valid)
